# baseline (speedup 1.0000x reference)
import functools

import jax
import jax.numpy as jnp
from jax import lax
from jax.experimental import pallas as pl
from jax.experimental.pallas import tpu as pltpu
from jax.experimental.pallas import tpu_sc as plsc


def _sc_gather(table, idx, n, k):
    info = plsc.get_sparse_core_info()
    nw = info.num_cores * info.num_subcores
    b_per_w = n // nw

    mesh = plsc.VectorSubcoreMesh(core_axis_name="c", subcore_axis_name="s")

    @functools.partial(
        pl.kernel,
        mesh=mesh,
        compiler_params=pltpu.CompilerParams(use_tc_tiling_on_sc=False),
        out_type=jax.ShapeDtypeStruct((n, k), jnp.float32),
        scratch_types=[
            pltpu.VMEM((b_per_w,), jnp.int32),
            pltpu.VMEM((b_per_w, k), jnp.float32),
            pltpu.SemaphoreType.DMA,
        ],
    )
    def gather_kernel(table_hbm, idx_hbm, out_hbm, idx_v, rows_v, sem):
        wid = lax.axis_index("s") * info.num_cores + lax.axis_index("c")
        base = wid * b_per_w
        pltpu.sync_copy(idx_hbm.at[pl.ds(base, b_per_w)], idx_v)
        pltpu.async_copy(table_hbm.at[idx_v], rows_v, sem).wait()
        pltpu.sync_copy(rows_v, out_hbm.at[pl.ds(base, b_per_w)])

    return gather_kernel(table, idx)


def _tc_matmul(g128, r_big, b, h, k, d, bb=32):
    pack = 128 // k
    n_lines = b * h // pack
    block_m = n_lines // 16

    def mm_body(g_ref, r_ref, o_ref):
        o_ref[...] = jnp.dot(g_ref[...], r_ref[...],
                             preferred_element_type=jnp.float32)

    out512 = pl.pallas_call(
        mm_body,
        grid=(n_lines // block_m,),
        in_specs=[
            pl.BlockSpec((block_m, 128), lambda i: (i, 0)),
            pl.BlockSpec((128, pack * d), lambda i: (0, 0)),
        ],
        out_specs=pl.BlockSpec((block_m, pack * d), lambda i: (i, 0)),
        out_shape=jax.ShapeDtypeStruct((n_lines, pack * d), jnp.float32),
    )(g128, r_big)
    return out512


def kernel(x, L, R):
    b, h = x.shape
    v, k = L.shape
    d, _ = R.shape
    pack = 128 // k

    h_pad = ((h + pack - 1) // pack) * pack
    n = b * h_pad

    idx = jnp.pad(x, ((0, 0), (0, h_pad - h))).reshape(n).astype(jnp.int32)
    g = _sc_gather(L, idx, n, k)
    g128 = g.reshape(n // pack, 128)
    r_big = jnp.kron(jnp.eye(pack, dtype=jnp.float32), R.T)
    out512 = _tc_matmul(g128, r_big, b, h_pad, k, d)
    out = out512.reshape(b, h_pad, d)
    if h_pad != h:
        out = lax.slice(out, (0, 0, 0), (b, h, d))
    return out


# speedup vs baseline: 1.1547x; 1.1547x over previous
import functools

import jax
import jax.numpy as jnp
from jax import lax
from jax.experimental import pallas as pl
from jax.experimental.pallas import tpu as pltpu
from jax.experimental.pallas import tpu_sc as plsc


def _sc_gather(table, idx, n, k):
    info = plsc.get_sparse_core_info()
    nw = info.num_cores * info.num_subcores
    b_per_w = n // nw

    mesh = plsc.VectorSubcoreMesh(core_axis_name="c", subcore_axis_name="s")

    @functools.partial(
        pl.kernel,
        mesh=mesh,
        compiler_params=pltpu.CompilerParams(use_tc_tiling_on_sc=False),
        out_type=jax.ShapeDtypeStruct((n, k), jnp.float32),
        scratch_types=[
            pltpu.VMEM((b_per_w,), jnp.int32),
            pltpu.VMEM((b_per_w, k), jnp.float32),
            pltpu.SemaphoreType.DMA,
        ],
    )
    def gather_kernel(table_hbm, idx_hbm, out_hbm, idx_v, rows_v, sem):
        wid = lax.axis_index("s") * info.num_cores + lax.axis_index("c")
        base = wid * b_per_w
        pltpu.sync_copy(idx_hbm.at[pl.ds(base, b_per_w)], idx_v)
        pltpu.async_copy(table_hbm.at[idx_v], rows_v, sem).wait()
        pltpu.sync_copy(rows_v, out_hbm.at[pl.ds(base, b_per_w)])

    return gather_kernel(table, idx)


def _tc_matmul(g128, r_big, n_lines, width):
    block_m = n_lines // 16

    def mm_body(g_ref, r_ref, o_ref):
        o_ref[...] = jnp.dot(g_ref[...], r_ref[...],
                             preferred_element_type=jnp.float32)

    return pl.pallas_call(
        mm_body,
        grid=(n_lines // block_m,),
        in_specs=[
            pl.BlockSpec((block_m, 128), lambda i: (i, 0)),
            pl.BlockSpec((128, width), lambda i: (0, 0)),
        ],
        out_specs=pl.BlockSpec((block_m, width), lambda i: (i, 0)),
        out_shape=jax.ShapeDtypeStruct((n_lines, width), jnp.float32),
    )(g128, r_big)


def _sc_relabel(src3d, b, h, d, bb=32):
    info = plsc.get_sparse_core_info()
    nw = info.num_cores * info.num_subcores
    b_per_w = b // nw

    mesh = plsc.VectorSubcoreMesh(core_axis_name="c", subcore_axis_name="s")

    @functools.partial(
        pl.kernel,
        mesh=mesh,
        compiler_params=pltpu.CompilerParams(use_tc_tiling_on_sc=False),
        out_type=jax.ShapeDtypeStruct((b, h, d), jnp.float32),
        scratch_types=[
            pltpu.VMEM((bb, h, d), jnp.float32),
        ],
    )
    def relabel_kernel(src_hbm, out_hbm, vbuf):
        wid = lax.axis_index("s") * info.num_cores + lax.axis_index("c")

        def body(i, _):
            base = wid * b_per_w + i * bb
            pltpu.sync_copy(src_hbm.at[pl.ds(base, bb)], vbuf)
            pltpu.sync_copy(vbuf, out_hbm.at[pl.ds(base, bb)])
            return _

        lax.fori_loop(0, b_per_w // bb, body, 0, unroll=False)

    return relabel_kernel(src3d)


def kernel(x, L, R):
    b, h = x.shape
    v, k = L.shape
    d, _ = R.shape
    n = b * h
    pack = 128 // k

    idx = x.reshape(n).astype(jnp.int32)
    g = _sc_gather(L, idx, n, k)
    g128 = g.reshape(n // pack, 128)
    r_big = jnp.kron(jnp.eye(pack, dtype=jnp.float32), R.T)
    out512 = _tc_matmul(g128, r_big, n // pack, pack * d)
    return _sc_relabel(out512.reshape(b, h, d), b, h, d)
